# Initial kernel scaffold; baseline (speedup 1.0000x reference)
import jax
import jax.numpy as jnp
from jax.experimental import pallas as pl


def kernel(x, edge_index, edge_attr, m1_w, m1_b, m2_w, m2_b, m3_w, m3_b, dist_w, p_vec_w):
    raise NotImplementedError("write your pallas kernel here")


# speedup vs baseline: 12.9268x; 12.9268x over previous
import functools

import jax
import jax.numpy as jnp
from jax import lax
from jax.experimental import pallas as pl
from jax.experimental.pallas import tpu as pltpu
from jax.experimental.pallas import tpu_sc as plsc

N = 10000
E = 320000
DIN = 128
DOUT = 16
H = 0.1
NSTEPS = 5

NC = 2
NS = 16
NW = NC * NS
EPT = E // NW
CW = 100
CH = EPT // CW
NT = 10240
RPT = NT // NW
RPS = NT // NS

_mesh = plsc.VectorSubcoreMesh(core_axis_name="c", subcore_axis_name="s")


def _mlp_body(x_ref, w1_ref, b1_ref, w2_ref, b2_ref, w3_ref, b3_ref,
              xo_ref, nxo_ref):
    h = jnp.dot(x_ref[...], w1_ref[...], preferred_element_type=jnp.float32)
    h = jnp.maximum(h + b1_ref[...], 0.0)
    h = jnp.dot(h, w2_ref[...], preferred_element_type=jnp.float32)
    h = jnp.maximum(h + b2_ref[...], 0.0)
    h = jnp.dot(h, w3_ref[...], preferred_element_type=jnp.float32)
    xo = jnp.abs(h + b3_ref[...])
    xo_ref[...] = xo
    nxo_ref[...] = -xo


def _mlp(x, w1t, b1, w2t, b2, w3t, b3):
    blk = 1000
    g = N // blk
    full = lambda shape: pl.BlockSpec(shape, lambda i: (0, 0))
    return pl.pallas_call(
        _mlp_body,
        grid=(g,),
        in_specs=[
            pl.BlockSpec((blk, DIN), lambda i: (i, 0)),
            full((DIN, 512)), full((1, 512)),
            full((512, DIN)), full((1, DIN)),
            full((DIN, DOUT)), full((1, DOUT)),
        ],
        out_specs=[pl.BlockSpec((blk, DOUT), lambda i: (i, 0)),
                   pl.BlockSpec((blk, DOUT), lambda i: (i, 0))],
        out_shape=[jax.ShapeDtypeStruct((N, DOUT), jnp.float32),
                   jax.ShapeDtypeStruct((N, DOUT), jnp.float32)],
    )(x, w1t, b1, w2t, b2, w3t, b3)


def _prep_inv_body(degp_ref, inv_ref):
    deg = degp_ref[0] + degp_ref[1]
    inv = jnp.where(deg > 0.0, 1.0 / deg, 0.0)
    inv_ref[...] = jnp.broadcast_to(inv[:, None], inv_ref.shape)


def _prep_inv(degp):
    g = 10
    nblk = NT // g
    return pl.pallas_call(
        _prep_inv_body,
        grid=(g,),
        in_specs=[pl.BlockSpec((NC, nblk), lambda i: (0, i))],
        out_specs=pl.BlockSpec((nblk, DOUT), lambda i: (i, 0)),
        out_shape=jax.ShapeDtypeStruct((NT, DOUT), jnp.float32),
    )(degp)


def _prep_sq_body(dist_ref, sq_ref):
    sq = jnp.sqrt(jax.nn.sigmoid(dist_ref[...]))
    sq_ref[...] = jnp.broadcast_to(sq[:, :, None], sq_ref.shape)


def _prep_sq(dist2d):
    return pl.pallas_call(
        _prep_sq_body,
        grid=(64,),
        in_specs=[pl.BlockSpec((8, 625), lambda i: (i, 0))],
        out_specs=pl.BlockSpec((8, 625, DOUT), lambda i: (i, 0, 0)),
        out_shape=jax.ShapeDtypeStruct((512, 625, DOUT), jnp.float32),
    )(dist2d)


@functools.partial(
    pl.kernel,
    out_type=jax.ShapeDtypeStruct((NC, NT), jnp.float32),
    mesh=_mesh,
    compiler_params=pltpu.CompilerParams(use_tc_tiling_on_sc=False),
    scratch_types=[
        pltpu.VMEM((CH, CW), jnp.int32),
        pltpu.VMEM((CH, CW), jnp.float32),
        pltpu.VMEM_SHARED((NT,), jnp.float32),
        pltpu.VMEM((RPS,), jnp.float32),
    ],
)
def _deg_sc(src_hbm, attr_hbm, out_hbm, src_v, attr_v, deg_sh, buf_v):
    c = lax.axis_index("c")
    s = lax.axis_index("s")
    wid = c * NS + s
    pltpu.sync_copy(src_hbm.at[wid], src_v)
    pltpu.sync_copy(attr_hbm.at[wid], attr_v)

    def z(i, carry):
        buf_v[pl.ds(i * 16, 16)] = jnp.zeros((16,), jnp.float32)
        return carry
    lax.fori_loop(0, RPS // 16, z, 0)
    pltpu.sync_copy(buf_v, deg_sh.at[pl.ds(s * RPS, RPS)])
    plsc.subcore_barrier()

    def body(ch, carry):
        pltpu.sync_copy(attr_v.at[ch], deg_sh.at[src_v.at[ch]], add=True)
        return carry
    lax.fori_loop(0, CH, body, 0)
    plsc.subcore_barrier()

    pltpu.sync_copy(deg_sh.at[pl.ds(s * RPS, RPS)], buf_v)
    pltpu.sync_copy(buf_v, out_hbm.at[c, pl.ds(s * RPS, RPS)])


@functools.partial(
    pl.kernel,
    out_type=jax.ShapeDtypeStruct((NC, NT, DOUT), jnp.float32),
    mesh=_mesh,
    compiler_params=pltpu.CompilerParams(use_tc_tiling_on_sc=False),
    scratch_types=[
        pltpu.VMEM((CH, CW), jnp.int32),
        pltpu.VMEM((CH, CW), jnp.int32),
        pltpu.VMEM((CW, DOUT), jnp.float32),
        pltpu.VMEM((CW, DOUT), jnp.float32),
        pltpu.VMEM((CW, DOUT), jnp.float32),
        pltpu.VMEM((CW, DOUT), jnp.float32),
        pltpu.VMEM_SHARED((NT, DOUT), jnp.float32),
        pltpu.VMEM((RPS, DOUT), jnp.float32),
        pltpu.SemaphoreType.DMA,
        pltpu.SemaphoreType.DMA,
    ],
)
def _stage_sc(yt_hbm, src_hbm, dst_hbm, sq_hbm, out_hbm,
              src_v, dst_v, sq_v, ys_v, yd_v, msg_v, gn_sh, buf_v,
              sem1, sem2):
    c = lax.axis_index("c")
    s = lax.axis_index("s")
    wid = c * NS + s
    pltpu.sync_copy(src_hbm.at[wid], src_v)
    pltpu.sync_copy(dst_hbm.at[wid], dst_v)

    def z(i, carry):
        buf_v[i] = jnp.zeros((DOUT,), jnp.float32)
        return carry
    lax.fori_loop(0, RPS, z, 0)
    pltpu.sync_copy(buf_v, gn_sh.at[pl.ds(s * RPS, RPS)])
    plsc.subcore_barrier()

    def chunk(ch, carry):
        g1 = pltpu.async_copy(yt_hbm.at[src_v.at[ch]], ys_v, sem1)
        g2 = pltpu.async_copy(yt_hbm.at[dst_v.at[ch]], yd_v, sem2)
        pltpu.sync_copy(sq_hbm.at[wid, ch], sq_v)
        g1.wait()
        g2.wait()

        def edge(e, c2):
            msg_v[e] = jnp.maximum(ys_v[e] - yd_v[e], 0.0) * sq_v[e]
            return c2
        lax.fori_loop(0, CW, edge, 0)
        pltpu.sync_copy(msg_v, gn_sh.at[src_v.at[ch]], add=True)
        return carry
    lax.fori_loop(0, CH, chunk, 0)
    plsc.subcore_barrier()

    pltpu.sync_copy(gn_sh.at[pl.ds(s * RPS, RPS)], buf_v)
    pltpu.sync_copy(buf_v, out_hbm.at[c, pl.ds(s * RPS, RPS)])


def _make_combine(coeffs, negate):
    nj = len(coeffs)

    def body(*refs):
        y_hbm = refs[0]
        inv_hbm = refs[1]
        p_hbm = refs[2:2 + nj]
        out_hbm = refs[2 + nj]
        y_v = refs[3 + nj]
        inv_v = refs[4 + nj]
        pa_v = refs[5 + nj:5 + 2 * nj]
        pb_v = refs[5 + 2 * nj:5 + 3 * nj]
        o_v = refs[5 + 3 * nj]

        c = lax.axis_index("c")
        s = lax.axis_index("s")
        wid = c * NS + s
        base = wid * RPT
        pltpu.sync_copy(y_hbm.at[pl.ds(base, RPT)], y_v)
        pltpu.sync_copy(inv_hbm.at[pl.ds(base, RPT)], inv_v)
        for j in range(nj):
            pltpu.sync_copy(p_hbm[j].at[0, pl.ds(base, RPT)], pa_v[j])
            pltpu.sync_copy(p_hbm[j].at[1, pl.ds(base, RPT)], pb_v[j])

        def row(i, carry):
            acc = y_v[i]
            inv = inv_v[i]
            for j in range(nj):
                k = 1.0 - (pa_v[j][i] + pb_v[j][i]) * inv
                acc = acc + coeffs[j] * k
            o_v[i] = -acc if negate else acc
            return carry
        lax.fori_loop(0, RPT, row, 0)
        pltpu.sync_copy(o_v, out_hbm.at[pl.ds(base, RPT)])

    scratch = ([pltpu.VMEM((RPT, DOUT), jnp.float32)] * (2 + 2 * nj)
               + [pltpu.VMEM((RPT, DOUT), jnp.float32)])
    return pl.kernel(
        body,
        out_type=jax.ShapeDtypeStruct((NT, DOUT), jnp.float32),
        mesh=_mesh,
        compiler_params=pltpu.CompilerParams(use_tc_tiling_on_sc=False),
        scratch_types=scratch,
    )


_comb_yt2 = _make_combine((H / 3.0,), False)
_comb_yt3 = _make_combine((-H / 3.0, H), False)
_comb_yt4 = _make_combine((H, -H, H), False)
_comb_y = _make_combine((H / 8.0, 3.0 * H / 8.0, 3.0 * H / 8.0, H / 8.0), False)
_comb_y_neg = _make_combine((H / 8.0, 3.0 * H / 8.0, 3.0 * H / 8.0, H / 8.0), True)


def kernel(x, edge_index, edge_attr, m1_w, m1_b, m2_w, m2_b, m3_w, m3_b,
           dist_w, p_vec_w):
    del p_vec_w
    src3 = edge_index[0].reshape(NW, CH, CW)
    dst3 = edge_index[1].reshape(NW, CH, CW)
    attr3 = edge_attr.reshape(NW, CH, CW)

    xo, neg_xo = _mlp(x, m1_w.T, m1_b.reshape(1, 512),
                      m2_w.T, m2_b.reshape(1, DIN),
                      m3_w.T, m3_b.reshape(1, DOUT))

    degp = _deg_sc(src3, attr3)
    inv16 = _prep_inv(degp)
    sq3 = _prep_sq(dist_w.reshape(512, 625)).reshape(NW, CH, CW, DOUT)

    y = jnp.pad(xo, ((0, NT - N), (0, 0)))
    for step in range(NSTEPS):
        p1 = _stage_sc(y, src3, dst3, sq3)
        yt2 = _comb_yt2(y, inv16, p1)
        p2 = _stage_sc(yt2, src3, dst3, sq3)
        yt3 = _comb_yt3(y, inv16, p1, p2)
        p3 = _stage_sc(yt3, src3, dst3, sq3)
        yt4 = _comb_yt4(y, inv16, p1, p2, p3)
        p4 = _stage_sc(yt4, src3, dst3, sq3)
        comb = _comb_y_neg if step == NSTEPS - 1 else _comb_y
        y = comb(y, inv16, p1, p2, p3, p4)

    return neg_xo, y[:N].T


# speedup vs baseline: 19.0459x; 1.4734x over previous
import functools

import jax
import jax.numpy as jnp
from jax import lax
from jax.experimental import pallas as pl
from jax.experimental.pallas import tpu as pltpu
from jax.experimental.pallas import tpu_sc as plsc

N = 10000
E = 320000
DIN = 128
DOUT = 16
H = 0.1
NSTEPS = 5

NC = 2
NS = 16
NW = NC * NS
EPT = E // NW
CW = 125
CH = EPT // CW
NT = 10240
RPT = NT // NW
RPS = NT // NS

_mesh = plsc.VectorSubcoreMesh(core_axis_name="c", subcore_axis_name="s")


def _mlp_body(x_ref, w1_ref, b1_ref, w2_ref, b2_ref, w3_ref, b3_ref,
              xo_ref, nxo_ref):
    h = jnp.dot(x_ref[...], w1_ref[...], preferred_element_type=jnp.float32)
    h = jnp.maximum(h + b1_ref[...], 0.0)
    h = jnp.dot(h, w2_ref[...], preferred_element_type=jnp.float32)
    h = jnp.maximum(h + b2_ref[...], 0.0)
    h = jnp.dot(h, w3_ref[...], preferred_element_type=jnp.float32)
    xo = jnp.abs(h + b3_ref[...])
    xo_ref[...] = xo
    nxo_ref[...] = -xo


def _mlp(x, w1t, b1, w2t, b2, w3t, b3):
    blk = 1000
    g = N // blk
    full = lambda shape: pl.BlockSpec(shape, lambda i: (0, 0))
    return pl.pallas_call(
        _mlp_body,
        grid=(g,),
        in_specs=[
            pl.BlockSpec((blk, DIN), lambda i: (i, 0)),
            full((DIN, 512)), full((1, 512)),
            full((512, DIN)), full((1, DIN)),
            full((DIN, DOUT)), full((1, DOUT)),
        ],
        out_specs=[pl.BlockSpec((blk, DOUT), lambda i: (i, 0)),
                   pl.BlockSpec((blk, DOUT), lambda i: (i, 0))],
        out_shape=[jax.ShapeDtypeStruct((N, DOUT), jnp.float32),
                   jax.ShapeDtypeStruct((N, DOUT), jnp.float32)],
    )(x, w1t, b1, w2t, b2, w3t, b3)


def _prep_inv_body(degp_ref, inv_ref):
    deg = degp_ref[0] + degp_ref[1]
    inv = jnp.where(deg > 0.0, 1.0 / deg, 0.0)
    inv_ref[...] = jnp.broadcast_to(inv[:, None], inv_ref.shape)


def _prep_inv(degp):
    g = 10
    nblk = NT // g
    return pl.pallas_call(
        _prep_inv_body,
        grid=(g,),
        in_specs=[pl.BlockSpec((NC, nblk), lambda i: (0, i))],
        out_specs=pl.BlockSpec((nblk, DOUT), lambda i: (i, 0)),
        out_shape=jax.ShapeDtypeStruct((NT, DOUT), jnp.float32),
    )(degp)


def _prep_sq_body(dist_ref, sq_ref):
    sq = jnp.sqrt(jax.nn.sigmoid(dist_ref[...]))
    sq_ref[...] = jnp.broadcast_to(sq[:, :, None], sq_ref.shape)


def _prep_sq(dist2d):
    return pl.pallas_call(
        _prep_sq_body,
        grid=(64,),
        in_specs=[pl.BlockSpec((8, 625), lambda i: (i, 0))],
        out_specs=pl.BlockSpec((8, 625, DOUT), lambda i: (i, 0, 0)),
        out_shape=jax.ShapeDtypeStruct((512, 625, DOUT), jnp.float32),
    )(dist2d)


@functools.partial(
    pl.kernel,
    out_type=jax.ShapeDtypeStruct((NC, NT), jnp.float32),
    mesh=_mesh,
    compiler_params=pltpu.CompilerParams(use_tc_tiling_on_sc=False),
    scratch_types=[
        pltpu.VMEM((CH, CW), jnp.int32),
        pltpu.VMEM((CH, CW), jnp.float32),
        pltpu.VMEM_SHARED((NT,), jnp.float32),
        pltpu.VMEM((RPS,), jnp.float32),
    ],
)
def _deg_sc(src_hbm, attr_hbm, out_hbm, src_v, attr_v, deg_sh, buf_v):
    c = lax.axis_index("c")
    s = lax.axis_index("s")
    wid = c * NS + s
    pltpu.sync_copy(src_hbm.at[wid], src_v)
    pltpu.sync_copy(attr_hbm.at[wid], attr_v)

    def z(i, carry):
        buf_v[pl.ds(i * 16, 16)] = jnp.zeros((16,), jnp.float32)
        return carry
    lax.fori_loop(0, RPS // 16, z, 0)
    pltpu.sync_copy(buf_v, deg_sh.at[pl.ds(s * RPS, RPS)])
    plsc.subcore_barrier()

    def body(ch, carry):
        pltpu.sync_copy(attr_v.at[ch], deg_sh.at[src_v.at[ch]], add=True)
        return carry
    lax.fori_loop(0, CH, body, 0)
    plsc.subcore_barrier()

    pltpu.sync_copy(deg_sh.at[pl.ds(s * RPS, RPS)], buf_v)
    pltpu.sync_copy(buf_v, out_hbm.at[c, pl.ds(s * RPS, RPS)])


@functools.partial(
    pl.kernel,
    out_type=jax.ShapeDtypeStruct((NC, NT, DOUT), jnp.float32),
    mesh=_mesh,
    compiler_params=pltpu.CompilerParams(use_tc_tiling_on_sc=False),
    scratch_types=[
        pltpu.VMEM((CH, CW), jnp.int32),
        pltpu.VMEM((CH, CW), jnp.int32),
        pltpu.VMEM((2, CW, DOUT), jnp.float32),
        pltpu.VMEM((2, CW, DOUT), jnp.float32),
        pltpu.VMEM((2, CW, DOUT), jnp.float32),
        pltpu.VMEM((2, CW, DOUT), jnp.float32),
        pltpu.VMEM_SHARED((NT, DOUT), jnp.float32),
        pltpu.VMEM((RPS, DOUT), jnp.float32),
        pltpu.SemaphoreType.DMA,
        pltpu.SemaphoreType.DMA,
        pltpu.SemaphoreType.DMA,
        pltpu.SemaphoreType.DMA,
    ],
)
def _stage_sc(yt_hbm, src_hbm, dst_hbm, sq_hbm, out_hbm,
              src_v, dst_v, sq_v, ys_v, yd_v, msg_v, gn_sh, buf_v,
              sem_s, sem_d, sem_q, sem_w):
    c = lax.axis_index("c")
    s = lax.axis_index("s")
    wid = c * NS + s
    pltpu.sync_copy(src_hbm.at[wid], src_v)
    pltpu.sync_copy(dst_hbm.at[wid], dst_v)

    def z(i, carry):
        buf_v[i] = jnp.zeros((DOUT,), jnp.float32)
        return carry
    lax.fori_loop(0, RPS, z, 0)
    pltpu.sync_copy(buf_v, gn_sh.at[pl.ds(s * RPS, RPS)])
    plsc.subcore_barrier()

    def fetch(ch, b):
        pltpu.async_copy(yt_hbm.at[src_v.at[ch]], ys_v.at[b], sem_s)
        pltpu.async_copy(yt_hbm.at[dst_v.at[ch]], yd_v.at[b], sem_d)
        pltpu.async_copy(sq_hbm.at[wid, ch], sq_v.at[b], sem_q)

    def wait_fetch(b):
        pltpu.make_async_copy(yt_hbm.at[src_v.at[0]], ys_v.at[b], sem_s).wait()
        pltpu.make_async_copy(yt_hbm.at[dst_v.at[0]], yd_v.at[b], sem_d).wait()
        pltpu.make_async_copy(sq_hbm.at[wid, 0], sq_v.at[b], sem_q).wait()

    def compute(ch, b):
        def edge(e, c2):
            msg_v[b, e] = (jnp.maximum(ys_v[b, e] - yd_v[b, e], 0.0)
                           * sq_v[b, e])
            return c2
        lax.fori_loop(0, CW, edge, 0, unroll=4)
        pltpu.async_copy(msg_v.at[b], gn_sh.at[src_v.at[ch]], sem_w, add=True)

    def wait_scatter(b):
        pltpu.make_async_copy(msg_v.at[b], gn_sh.at[src_v.at[0]], sem_w).wait()

    fetch(0, 0)

    def pair(i, carry):
        ch0 = 2 * i
        wait_fetch(0)
        fetch(ch0 + 1, 1)

        @pl.when(i > 0)
        def _():
            wait_scatter(0)
        compute(ch0, 0)

        wait_fetch(1)

        @pl.when(i + 1 < CH // 2)
        def _():
            fetch(ch0 + 2, 0)

        @pl.when(i > 0)
        def _():
            wait_scatter(1)
        compute(ch0 + 1, 1)
        return carry
    lax.fori_loop(0, CH // 2, pair, 0)
    wait_scatter(0)
    wait_scatter(1)
    plsc.subcore_barrier()

    pltpu.sync_copy(gn_sh.at[pl.ds(s * RPS, RPS)], buf_v)
    pltpu.sync_copy(buf_v, out_hbm.at[c, pl.ds(s * RPS, RPS)])


def _make_combine(coeffs, negate):
    nj = len(coeffs)

    def body(*refs):
        y_hbm = refs[0]
        inv_hbm = refs[1]
        p_hbm = refs[2:2 + nj]
        out_hbm = refs[2 + nj]
        y_v = refs[3 + nj]
        inv_v = refs[4 + nj]
        pa_v = refs[5 + nj:5 + 2 * nj]
        pb_v = refs[5 + 2 * nj:5 + 3 * nj]
        o_v = refs[5 + 3 * nj]

        c = lax.axis_index("c")
        s = lax.axis_index("s")
        wid = c * NS + s
        base = wid * RPT
        pltpu.sync_copy(y_hbm.at[pl.ds(base, RPT)], y_v)
        pltpu.sync_copy(inv_hbm.at[pl.ds(base, RPT)], inv_v)
        for j in range(nj):
            pltpu.sync_copy(p_hbm[j].at[0, pl.ds(base, RPT)], pa_v[j])
            pltpu.sync_copy(p_hbm[j].at[1, pl.ds(base, RPT)], pb_v[j])

        def row(i, carry):
            acc = y_v[i]
            inv = inv_v[i]
            for j in range(nj):
                k = 1.0 - (pa_v[j][i] + pb_v[j][i]) * inv
                acc = acc + coeffs[j] * k
            o_v[i] = -acc if negate else acc
            return carry
        lax.fori_loop(0, RPT, row, 0)
        pltpu.sync_copy(o_v, out_hbm.at[pl.ds(base, RPT)])

    scratch = ([pltpu.VMEM((RPT, DOUT), jnp.float32)] * (2 + 2 * nj)
               + [pltpu.VMEM((RPT, DOUT), jnp.float32)])
    return pl.kernel(
        body,
        out_type=jax.ShapeDtypeStruct((NT, DOUT), jnp.float32),
        mesh=_mesh,
        compiler_params=pltpu.CompilerParams(use_tc_tiling_on_sc=False),
        scratch_types=scratch,
    )


_comb_yt2 = _make_combine((H / 3.0,), False)
_comb_yt3 = _make_combine((-H / 3.0, H), False)
_comb_yt4 = _make_combine((H, -H, H), False)
_comb_y = _make_combine((H / 8.0, 3.0 * H / 8.0, 3.0 * H / 8.0, H / 8.0), False)
_comb_y_neg = _make_combine((H / 8.0, 3.0 * H / 8.0, 3.0 * H / 8.0, H / 8.0), True)


def kernel(x, edge_index, edge_attr, m1_w, m1_b, m2_w, m2_b, m3_w, m3_b,
           dist_w, p_vec_w):
    del p_vec_w
    src3 = edge_index[0].reshape(NW, CH, CW)
    dst3 = edge_index[1].reshape(NW, CH, CW)
    attr3 = edge_attr.reshape(NW, CH, CW)

    xo, neg_xo = _mlp(x, m1_w.T, m1_b.reshape(1, 512),
                      m2_w.T, m2_b.reshape(1, DIN),
                      m3_w.T, m3_b.reshape(1, DOUT))

    degp = _deg_sc(src3, attr3)
    inv16 = _prep_inv(degp)
    sq3 = _prep_sq(dist_w.reshape(512, 625)).reshape(NW, CH, CW, DOUT)

    y = jnp.pad(xo, ((0, NT - N), (0, 0)))
    for step in range(NSTEPS):
        p1 = _stage_sc(y, src3, dst3, sq3)
        yt2 = _comb_yt2(y, inv16, p1)
        p2 = _stage_sc(yt2, src3, dst3, sq3)
        yt3 = _comb_yt3(y, inv16, p1, p2)
        p3 = _stage_sc(yt3, src3, dst3, sq3)
        yt4 = _comb_yt4(y, inv16, p1, p2, p3)
        p4 = _stage_sc(yt4, src3, dst3, sq3)
        comb = _comb_y_neg if step == NSTEPS - 1 else _comb_y
        y = comb(y, inv16, p1, p2, p3, p4)

    return neg_xo, y[:N].T


# speedup vs baseline: 19.1294x; 1.0044x over previous
import functools

import jax
import jax.numpy as jnp
from jax import lax
from jax.experimental import pallas as pl
from jax.experimental.pallas import tpu as pltpu
from jax.experimental.pallas import tpu_sc as plsc

N = 10000
E = 320000
DIN = 128
DOUT = 16
H = 0.1
NSTEPS = 5

NC = 2
NS = 16
NW = NC * NS
EPT = E // NW
CW = 125
CH = EPT // CW
NT = 10240
RPT = NT // NW
RPS = NT // NS

_mesh = plsc.VectorSubcoreMesh(core_axis_name="c", subcore_axis_name="s")


def _mlp_body(x_ref, w1_ref, b1_ref, w2_ref, b2_ref, w3_ref, b3_ref,
              xo_ref, nxo_ref):
    h = jnp.dot(x_ref[...], w1_ref[...], preferred_element_type=jnp.float32)
    h = jnp.maximum(h + b1_ref[...], 0.0)
    h = jnp.dot(h, w2_ref[...], preferred_element_type=jnp.float32)
    h = jnp.maximum(h + b2_ref[...], 0.0)
    h = jnp.dot(h, w3_ref[...], preferred_element_type=jnp.float32)
    xo = jnp.abs(h + b3_ref[...])
    xo_ref[...] = xo
    nxo_ref[...] = -xo


def _mlp(x, w1t, b1, w2t, b2, w3t, b3):
    blk = 1000
    g = N // blk
    full = lambda shape: pl.BlockSpec(shape, lambda i: (0, 0))
    return pl.pallas_call(
        _mlp_body,
        grid=(g,),
        in_specs=[
            pl.BlockSpec((blk, DIN), lambda i: (i, 0)),
            full((DIN, 512)), full((1, 512)),
            full((512, DIN)), full((1, DIN)),
            full((DIN, DOUT)), full((1, DOUT)),
        ],
        out_specs=[pl.BlockSpec((blk, DOUT), lambda i: (i, 0)),
                   pl.BlockSpec((blk, DOUT), lambda i: (i, 0))],
        out_shape=[jax.ShapeDtypeStruct((N, DOUT), jnp.float32),
                   jax.ShapeDtypeStruct((N, DOUT), jnp.float32)],
    )(x, w1t, b1, w2t, b2, w3t, b3)


def _prep_inv_body(degp_ref, inv_ref):
    deg = degp_ref[0] + degp_ref[1]
    inv = jnp.where(deg > 0.0, 1.0 / deg, 0.0)
    inv_ref[...] = jnp.broadcast_to(inv[:, None], inv_ref.shape)


def _prep_inv(degp):
    g = 10
    nblk = NT // g
    return pl.pallas_call(
        _prep_inv_body,
        grid=(g,),
        in_specs=[pl.BlockSpec((NC, nblk), lambda i: (0, i))],
        out_specs=pl.BlockSpec((nblk, DOUT), lambda i: (i, 0)),
        out_shape=jax.ShapeDtypeStruct((NT, DOUT), jnp.float32),
    )(degp)


def _prep_sq_body(dist_ref, sq_ref):
    sq = jnp.sqrt(jax.nn.sigmoid(dist_ref[...]))
    sq_ref[...] = jnp.broadcast_to(sq[:, :, None], sq_ref.shape)


def _prep_sq(dist2d):
    return pl.pallas_call(
        _prep_sq_body,
        grid=(64,),
        in_specs=[pl.BlockSpec((8, 625), lambda i: (i, 0))],
        out_specs=pl.BlockSpec((8, 625, DOUT), lambda i: (i, 0, 0)),
        out_shape=jax.ShapeDtypeStruct((512, 625, DOUT), jnp.float32),
    )(dist2d)


@functools.partial(
    pl.kernel,
    out_type=jax.ShapeDtypeStruct((NC, NT), jnp.float32),
    mesh=_mesh,
    compiler_params=pltpu.CompilerParams(use_tc_tiling_on_sc=False),
    scratch_types=[
        pltpu.VMEM((CH, CW), jnp.int32),
        pltpu.VMEM((CH, CW), jnp.float32),
        pltpu.VMEM_SHARED((NT,), jnp.float32),
        pltpu.VMEM((RPS,), jnp.float32),
    ],
)
def _deg_sc(src_hbm, attr_hbm, out_hbm, src_v, attr_v, deg_sh, buf_v):
    c = lax.axis_index("c")
    s = lax.axis_index("s")
    wid = c * NS + s
    pltpu.sync_copy(src_hbm.at[wid], src_v)
    pltpu.sync_copy(attr_hbm.at[wid], attr_v)

    def z(i, carry):
        buf_v[pl.ds(i * 16, 16)] = jnp.zeros((16,), jnp.float32)
        return carry
    lax.fori_loop(0, RPS // 16, z, 0)
    pltpu.sync_copy(buf_v, deg_sh.at[pl.ds(s * RPS, RPS)])
    plsc.subcore_barrier()

    def body(ch, carry):
        pltpu.sync_copy(attr_v.at[ch], deg_sh.at[src_v.at[ch]], add=True)
        return carry
    lax.fori_loop(0, CH, body, 0)
    plsc.subcore_barrier()

    pltpu.sync_copy(deg_sh.at[pl.ds(s * RPS, RPS)], buf_v)
    pltpu.sync_copy(buf_v, out_hbm.at[c, pl.ds(s * RPS, RPS)])


@functools.partial(
    pl.kernel,
    out_type=jax.ShapeDtypeStruct((NC, NT, DOUT), jnp.float32),
    mesh=_mesh,
    compiler_params=pltpu.CompilerParams(use_tc_tiling_on_sc=False),
    scratch_types=[
        pltpu.VMEM((CH, CW), jnp.int32),
        pltpu.VMEM((CH, CW), jnp.int32),
        pltpu.VMEM((4, CW, DOUT), jnp.float32),
        pltpu.VMEM((4, CW, DOUT), jnp.float32),
        pltpu.VMEM((4, CW, DOUT), jnp.float32),
        pltpu.VMEM((4, CW, DOUT), jnp.float32),
        pltpu.VMEM_SHARED((NT, DOUT), jnp.float32),
        pltpu.VMEM((RPS, DOUT), jnp.float32),
        pltpu.SemaphoreType.DMA,
        pltpu.SemaphoreType.DMA,
        pltpu.SemaphoreType.DMA,
        pltpu.SemaphoreType.DMA,
    ],
)
def _stage_sc(yt_hbm, src_hbm, dst_hbm, sq_hbm, out_hbm,
              src_v, dst_v, sq_v, ys_v, yd_v, msg_v, gn_sh, buf_v,
              sem_s, sem_d, sem_q, sem_w):
    c = lax.axis_index("c")
    s = lax.axis_index("s")
    wid = c * NS + s
    pltpu.sync_copy(src_hbm.at[wid], src_v)
    pltpu.sync_copy(dst_hbm.at[wid], dst_v)

    def z(i, carry):
        buf_v[i] = jnp.zeros((DOUT,), jnp.float32)
        return carry
    lax.fori_loop(0, RPS, z, 0)
    pltpu.sync_copy(buf_v, gn_sh.at[pl.ds(s * RPS, RPS)])
    plsc.subcore_barrier()

    def fetch(ch, b):
        pltpu.async_copy(yt_hbm.at[src_v.at[ch]], ys_v.at[b], sem_s)
        pltpu.async_copy(yt_hbm.at[dst_v.at[ch]], yd_v.at[b], sem_d)
        pltpu.async_copy(sq_hbm.at[wid, ch], sq_v.at[b], sem_q)

    def wait_fetch(b):
        pltpu.make_async_copy(yt_hbm.at[src_v.at[0]], ys_v.at[b], sem_s).wait()
        pltpu.make_async_copy(yt_hbm.at[dst_v.at[0]], yd_v.at[b], sem_d).wait()
        pltpu.make_async_copy(sq_hbm.at[wid, 0], sq_v.at[b], sem_q).wait()

    def compute(ch, b):
        def edge(e, c2):
            msg_v[b, e] = (jnp.maximum(ys_v[b, e] - yd_v[b, e], 0.0)
                           * sq_v[b, e])
            return c2
        lax.fori_loop(0, CW, edge, 0, unroll=5)
        pltpu.async_copy(msg_v.at[b], gn_sh.at[src_v.at[ch]], sem_w, add=True)

    def wait_scatter(b):
        pltpu.make_async_copy(msg_v.at[b], gn_sh.at[src_v.at[0]], sem_w).wait()

    NB = 4
    for b in range(NB):
        fetch(b, b)

    def ring(i, carry):
        base = i * NB
        for b in range(NB):
            wait_fetch(b)

            @pl.when(i > 0)
            def _():
                wait_scatter(b)
            compute(base + b, b)

            @pl.when(base + b + NB < CH)
            def _():
                fetch(base + b + NB, b)
        return carry
    lax.fori_loop(0, CH // NB, ring, 0)
    for b in range(NB):
        wait_scatter(b)
    plsc.subcore_barrier()

    pltpu.sync_copy(gn_sh.at[pl.ds(s * RPS, RPS)], buf_v)
    pltpu.sync_copy(buf_v, out_hbm.at[c, pl.ds(s * RPS, RPS)])


def _make_combine(coeffs, negate):
    nj = len(coeffs)

    def body(*refs):
        y_hbm = refs[0]
        inv_hbm = refs[1]
        p_hbm = refs[2:2 + nj]
        out_hbm = refs[2 + nj]
        y_v = refs[3 + nj]
        inv_v = refs[4 + nj]
        pa_v = refs[5 + nj:5 + 2 * nj]
        pb_v = refs[5 + 2 * nj:5 + 3 * nj]
        o_v = refs[5 + 3 * nj]

        c = lax.axis_index("c")
        s = lax.axis_index("s")
        wid = c * NS + s
        base = wid * RPT
        pltpu.sync_copy(y_hbm.at[pl.ds(base, RPT)], y_v)
        pltpu.sync_copy(inv_hbm.at[pl.ds(base, RPT)], inv_v)
        for j in range(nj):
            pltpu.sync_copy(p_hbm[j].at[0, pl.ds(base, RPT)], pa_v[j])
            pltpu.sync_copy(p_hbm[j].at[1, pl.ds(base, RPT)], pb_v[j])

        def row(i, carry):
            acc = y_v[i]
            inv = inv_v[i]
            for j in range(nj):
                k = 1.0 - (pa_v[j][i] + pb_v[j][i]) * inv
                acc = acc + coeffs[j] * k
            o_v[i] = -acc if negate else acc
            return carry
        lax.fori_loop(0, RPT, row, 0)
        pltpu.sync_copy(o_v, out_hbm.at[pl.ds(base, RPT)])

    scratch = ([pltpu.VMEM((RPT, DOUT), jnp.float32)] * (2 + 2 * nj)
               + [pltpu.VMEM((RPT, DOUT), jnp.float32)])
    return pl.kernel(
        body,
        out_type=jax.ShapeDtypeStruct((NT, DOUT), jnp.float32),
        mesh=_mesh,
        compiler_params=pltpu.CompilerParams(use_tc_tiling_on_sc=False),
        scratch_types=scratch,
    )


_comb_yt2 = _make_combine((H / 3.0,), False)
_comb_yt3 = _make_combine((-H / 3.0, H), False)
_comb_yt4 = _make_combine((H, -H, H), False)
_comb_y = _make_combine((H / 8.0, 3.0 * H / 8.0, 3.0 * H / 8.0, H / 8.0), False)
_comb_y_neg = _make_combine((H / 8.0, 3.0 * H / 8.0, 3.0 * H / 8.0, H / 8.0), True)


def kernel(x, edge_index, edge_attr, m1_w, m1_b, m2_w, m2_b, m3_w, m3_b,
           dist_w, p_vec_w):
    del p_vec_w
    src3 = edge_index[0].reshape(NW, CH, CW)
    dst3 = edge_index[1].reshape(NW, CH, CW)
    attr3 = edge_attr.reshape(NW, CH, CW)

    xo, neg_xo = _mlp(x, m1_w.T, m1_b.reshape(1, 512),
                      m2_w.T, m2_b.reshape(1, DIN),
                      m3_w.T, m3_b.reshape(1, DOUT))

    degp = _deg_sc(src3, attr3)
    inv16 = _prep_inv(degp)
    sq3 = _prep_sq(dist_w.reshape(512, 625)).reshape(NW, CH, CW, DOUT)

    y = jnp.pad(xo, ((0, NT - N), (0, 0)))
    for step in range(NSTEPS):
        p1 = _stage_sc(y, src3, dst3, sq3)
        yt2 = _comb_yt2(y, inv16, p1)
        p2 = _stage_sc(yt2, src3, dst3, sq3)
        yt3 = _comb_yt3(y, inv16, p1, p2)
        p3 = _stage_sc(yt3, src3, dst3, sq3)
        yt4 = _comb_yt4(y, inv16, p1, p2, p3)
        p4 = _stage_sc(yt4, src3, dst3, sq3)
        comb = _comb_y_neg if step == NSTEPS - 1 else _comb_y
        y = comb(y, inv16, p1, p2, p3, p4)

    return neg_xo, y[:N].T


# speedup vs baseline: 29.3921x; 1.5365x over previous
import functools

import jax
import jax.numpy as jnp
from jax import lax
from jax.experimental import pallas as pl
from jax.experimental.pallas import tpu as pltpu
from jax.experimental.pallas import tpu_sc as plsc

N = 10000
E = 320000
DIN = 128
DOUT = 16
H = 0.1
NSTEPS = 5

NC = 2
NS = 16
NW = NC * NS
EPT = E // NW
CW = 125
CH = EPT // CW
NT = 10240
RPT = NT // NW
RPS = NT // NS

_mesh = plsc.VectorSubcoreMesh(core_axis_name="c", subcore_axis_name="s")


def _mlp_body(x_ref, w1_ref, b1_ref, w2_ref, b2_ref, w3_ref, b3_ref,
              xo_ref, nxo_ref):
    h = jnp.dot(x_ref[...], w1_ref[...], preferred_element_type=jnp.float32)
    h = jnp.maximum(h + b1_ref[...], 0.0)
    h = jnp.dot(h, w2_ref[...], preferred_element_type=jnp.float32)
    h = jnp.maximum(h + b2_ref[...], 0.0)
    h = jnp.dot(h, w3_ref[...], preferred_element_type=jnp.float32)
    xo = jnp.abs(h + b3_ref[...])
    xo_ref[...] = xo
    nxo_ref[...] = -xo


def _mlp(x, w1t, b1, w2t, b2, w3t, b3):
    blk = 1000
    g = N // blk
    full = lambda shape: pl.BlockSpec(shape, lambda i: (0, 0))
    return pl.pallas_call(
        _mlp_body,
        grid=(g,),
        in_specs=[
            pl.BlockSpec((blk, DIN), lambda i: (i, 0)),
            full((DIN, 512)), full((1, 512)),
            full((512, DIN)), full((1, DIN)),
            full((DIN, DOUT)), full((1, DOUT)),
        ],
        out_specs=[pl.BlockSpec((blk, DOUT), lambda i: (i, 0)),
                   pl.BlockSpec((blk, DOUT), lambda i: (i, 0))],
        out_shape=[jax.ShapeDtypeStruct((N, DOUT), jnp.float32),
                   jax.ShapeDtypeStruct((N, DOUT), jnp.float32)],
    )(x, w1t, b1, w2t, b2, w3t, b3)


def _prep_inv_body(degp_ref, inv_ref):
    deg = degp_ref[0] + degp_ref[1]
    inv = jnp.where(deg > 0.0, 1.0 / deg, 0.0)
    inv_ref[...] = jnp.broadcast_to(inv[:, None], inv_ref.shape)


def _prep_inv(degp):
    g = 10
    nblk = NT // g
    return pl.pallas_call(
        _prep_inv_body,
        grid=(g,),
        in_specs=[pl.BlockSpec((NC, nblk), lambda i: (0, i))],
        out_specs=pl.BlockSpec((nblk, DOUT), lambda i: (i, 0)),
        out_shape=jax.ShapeDtypeStruct((NT, DOUT), jnp.float32),
    )(degp)


def _prep_sq_body(dist_ref, sq_ref):
    sq = jnp.sqrt(jax.nn.sigmoid(dist_ref[...]))
    sq_ref[...] = jnp.broadcast_to(sq[:, :, None], sq_ref.shape)


def _prep_sq(dist2d):
    return pl.pallas_call(
        _prep_sq_body,
        grid=(64,),
        in_specs=[pl.BlockSpec((8, 625), lambda i: (i, 0))],
        out_specs=pl.BlockSpec((8, 625, DOUT), lambda i: (i, 0, 0)),
        out_shape=jax.ShapeDtypeStruct((512, 625, DOUT), jnp.float32),
    )(dist2d)


@functools.partial(
    pl.kernel,
    out_type=jax.ShapeDtypeStruct((NC, NT), jnp.float32),
    mesh=_mesh,
    compiler_params=pltpu.CompilerParams(use_tc_tiling_on_sc=False),
    scratch_types=[
        pltpu.VMEM((CH, CW), jnp.int32),
        pltpu.VMEM((CH, CW), jnp.float32),
        pltpu.VMEM_SHARED((NT,), jnp.float32),
        pltpu.VMEM((RPS,), jnp.float32),
    ],
)
def _deg_sc(src_hbm, attr_hbm, out_hbm, src_v, attr_v, deg_sh, buf_v):
    c = lax.axis_index("c")
    s = lax.axis_index("s")
    wid = c * NS + s
    pltpu.sync_copy(src_hbm.at[wid], src_v)
    pltpu.sync_copy(attr_hbm.at[wid], attr_v)

    def z(i, carry):
        buf_v[pl.ds(i * 16, 16)] = jnp.zeros((16,), jnp.float32)
        return carry
    lax.fori_loop(0, RPS // 16, z, 0)
    pltpu.sync_copy(buf_v, deg_sh.at[pl.ds(s * RPS, RPS)])
    plsc.subcore_barrier()

    def body(ch, carry):
        pltpu.sync_copy(attr_v.at[ch], deg_sh.at[src_v.at[ch]], add=True)
        return carry
    lax.fori_loop(0, CH, body, 0)
    plsc.subcore_barrier()

    pltpu.sync_copy(deg_sh.at[pl.ds(s * RPS, RPS)], buf_v)
    pltpu.sync_copy(buf_v, out_hbm.at[c, pl.ds(s * RPS, RPS)])


@functools.partial(
    pl.kernel,
    out_type=jax.ShapeDtypeStruct((NC, NT, DOUT), jnp.float32),
    mesh=_mesh,
    compiler_params=pltpu.CompilerParams(use_tc_tiling_on_sc=False),
    scratch_types=[
        pltpu.VMEM((CH, CW), jnp.int32),
        pltpu.VMEM((CH, CW), jnp.int32),
        pltpu.VMEM((4, CW, DOUT), jnp.float32),
        pltpu.VMEM((4, CW, DOUT), jnp.float32),
        pltpu.VMEM((4, CW, DOUT), jnp.float32),
        pltpu.VMEM((4, CW, DOUT), jnp.float32),
        pltpu.VMEM_SHARED((NT, DOUT), jnp.float32),
        pltpu.VMEM((RPS, DOUT), jnp.float32),
        pltpu.SemaphoreType.DMA,
        pltpu.SemaphoreType.DMA,
        pltpu.SemaphoreType.DMA,
        pltpu.SemaphoreType.DMA,
    ],
)
def _stage_sc(yt_hbm, src_hbm, dst_hbm, sq_hbm, out_hbm,
              src_v, dst_v, sq_v, ys_v, yd_v, msg_v, gn_sh, buf_v,
              sem_s, sem_d, sem_q, sem_w):
    c = lax.axis_index("c")
    s = lax.axis_index("s")
    wid = c * NS + s
    pltpu.sync_copy(src_hbm.at[wid], src_v)
    pltpu.sync_copy(dst_hbm.at[wid], dst_v)

    def z(i, carry):
        buf_v[i] = jnp.zeros((DOUT,), jnp.float32)
        return carry
    lax.fori_loop(0, RPS, z, 0)
    pltpu.sync_copy(buf_v, gn_sh.at[pl.ds(s * RPS, RPS)])
    plsc.subcore_barrier()

    def fetch(ch, b):
        pltpu.async_copy(yt_hbm.at[src_v.at[ch]], ys_v.at[b], sem_s)
        pltpu.async_copy(yt_hbm.at[dst_v.at[ch]], yd_v.at[b], sem_d)
        pltpu.async_copy(sq_hbm.at[wid, ch], sq_v.at[b], sem_q)

    def wait_fetch(b):
        pltpu.make_async_copy(yt_hbm.at[src_v.at[0]], ys_v.at[b], sem_s).wait()
        pltpu.make_async_copy(yt_hbm.at[dst_v.at[0]], yd_v.at[b], sem_d).wait()
        pltpu.make_async_copy(sq_hbm.at[wid, 0], sq_v.at[b], sem_q).wait()

    def compute(ch, b):
        @functools.partial(plsc.parallel_loop, 0, CW, unroll=5)
        def edge(e):
            msg_v[b, e] = (jnp.maximum(ys_v[b, e] - yd_v[b, e], 0.0)
                           * sq_v[b, e])
        pltpu.async_copy(msg_v.at[b], gn_sh.at[src_v.at[ch]], sem_w, add=True)

    def wait_scatter(b):
        pltpu.make_async_copy(msg_v.at[b], gn_sh.at[src_v.at[0]], sem_w).wait()

    NB = 4
    for b in range(NB):
        fetch(b, b)

    def ring(i, carry):
        base = i * NB
        for b in range(NB):
            wait_fetch(b)

            @pl.when(i > 0)
            def _():
                wait_scatter(b)
            compute(base + b, b)

            @pl.when(base + b + NB < CH)
            def _():
                fetch(base + b + NB, b)
        return carry
    lax.fori_loop(0, CH // NB, ring, 0)
    for b in range(NB):
        wait_scatter(b)
    plsc.subcore_barrier()

    pltpu.sync_copy(gn_sh.at[pl.ds(s * RPS, RPS)], buf_v)
    pltpu.sync_copy(buf_v, out_hbm.at[c, pl.ds(s * RPS, RPS)])


def _make_combine(coeffs, negate):
    nj = len(coeffs)

    def body(*refs):
        y_hbm = refs[0]
        inv_hbm = refs[1]
        p_hbm = refs[2:2 + nj]
        out_hbm = refs[2 + nj]
        y_v = refs[3 + nj]
        inv_v = refs[4 + nj]
        pa_v = refs[5 + nj:5 + 2 * nj]
        pb_v = refs[5 + 2 * nj:5 + 3 * nj]
        o_v = refs[5 + 3 * nj]

        c = lax.axis_index("c")
        s = lax.axis_index("s")
        wid = c * NS + s
        base = wid * RPT
        pltpu.sync_copy(y_hbm.at[pl.ds(base, RPT)], y_v)
        pltpu.sync_copy(inv_hbm.at[pl.ds(base, RPT)], inv_v)
        for j in range(nj):
            pltpu.sync_copy(p_hbm[j].at[0, pl.ds(base, RPT)], pa_v[j])
            pltpu.sync_copy(p_hbm[j].at[1, pl.ds(base, RPT)], pb_v[j])

        def row(i, carry):
            acc = y_v[i]
            inv = inv_v[i]
            for j in range(nj):
                k = 1.0 - (pa_v[j][i] + pb_v[j][i]) * inv
                acc = acc + coeffs[j] * k
            o_v[i] = -acc if negate else acc
            return carry
        lax.fori_loop(0, RPT, row, 0)
        pltpu.sync_copy(o_v, out_hbm.at[pl.ds(base, RPT)])

    scratch = ([pltpu.VMEM((RPT, DOUT), jnp.float32)] * (2 + 2 * nj)
               + [pltpu.VMEM((RPT, DOUT), jnp.float32)])
    return pl.kernel(
        body,
        out_type=jax.ShapeDtypeStruct((NT, DOUT), jnp.float32),
        mesh=_mesh,
        compiler_params=pltpu.CompilerParams(use_tc_tiling_on_sc=False),
        scratch_types=scratch,
    )


_comb_yt2 = _make_combine((H / 3.0,), False)
_comb_yt3 = _make_combine((-H / 3.0, H), False)
_comb_yt4 = _make_combine((H, -H, H), False)
_comb_y = _make_combine((H / 8.0, 3.0 * H / 8.0, 3.0 * H / 8.0, H / 8.0), False)
_comb_y_neg = _make_combine((H / 8.0, 3.0 * H / 8.0, 3.0 * H / 8.0, H / 8.0), True)


def kernel(x, edge_index, edge_attr, m1_w, m1_b, m2_w, m2_b, m3_w, m3_b,
           dist_w, p_vec_w):
    del p_vec_w
    src3 = edge_index[0].reshape(NW, CH, CW)
    dst3 = edge_index[1].reshape(NW, CH, CW)
    attr3 = edge_attr.reshape(NW, CH, CW)

    xo, neg_xo = _mlp(x, m1_w.T, m1_b.reshape(1, 512),
                      m2_w.T, m2_b.reshape(1, DIN),
                      m3_w.T, m3_b.reshape(1, DOUT))

    degp = _deg_sc(src3, attr3)
    inv16 = _prep_inv(degp)
    sq3 = _prep_sq(dist_w.reshape(512, 625)).reshape(NW, CH, CW, DOUT)

    y = jnp.pad(xo, ((0, NT - N), (0, 0)))
    for step in range(NSTEPS):
        p1 = _stage_sc(y, src3, dst3, sq3)
        yt2 = _comb_yt2(y, inv16, p1)
        p2 = _stage_sc(yt2, src3, dst3, sq3)
        yt3 = _comb_yt3(y, inv16, p1, p2)
        p3 = _stage_sc(yt3, src3, dst3, sq3)
        yt4 = _comb_yt4(y, inv16, p1, p2, p3)
        p4 = _stage_sc(yt4, src3, dst3, sq3)
        comb = _comb_y_neg if step == NSTEPS - 1 else _comb_y
        y = comb(y, inv16, p1, p2, p3, p4)

    return neg_xo, y[:N].T
